# baseline (speedup 1.0000x reference)
import jax
import jax.numpy as jnp
from jax.experimental import pallas as pl

_N_CLASSES = 128
_ROWS = 16384
_COLS = 26
_BLOCK = 2048


def _onehot_body(x_ref, o_ref):
    j = pl.program_id(1)
    xf = x_ref[...].astype(jnp.bfloat16)
    jid = jax.lax.broadcasted_iota(jnp.int32, (_COLS, _N_CLASSES), 0)
    ej = (jid == j).astype(jnp.bfloat16)
    xrep = jax.lax.dot_general(
        xf, ej,
        dimension_numbers=(((1,), (0,)), ((), ())),
        preferred_element_type=jnp.float32,
    )
    kconst = jax.lax.broadcasted_iota(
        jnp.int32, (_BLOCK, _N_CLASSES), 1).astype(jnp.float32)
    o_ref[...] = (xrep == kconst).astype(jnp.int32)


def kernel(x):
    nb = _ROWS // _BLOCK
    out2d = pl.pallas_call(
        _onehot_body,
        grid=(nb, _COLS),
        in_specs=[pl.BlockSpec((_BLOCK, _COLS), lambda i, j: (i, 0))],
        out_specs=pl.BlockSpec(
            (_BLOCK, _N_CLASSES), lambda i, j: (j * nb + i, 0)),
        out_shape=jax.ShapeDtypeStruct((_COLS * _ROWS, _N_CLASSES), jnp.int32),
    )(x)
    return jnp.transpose(
        out2d.reshape(_COLS, _ROWS, _N_CLASSES), (1, 0, 2))


# speedup vs baseline: 1.9152x; 1.9152x over previous
import jax
import jax.numpy as jnp
from jax.experimental import pallas as pl

_N_CLASSES = 128
_ROWS = 16384
_COLS = 26
_W = _COLS * _N_CLASSES
_BLOCK = 1024


def _onehot_body(x_ref, o_ref):
    xf = x_ref[...].astype(jnp.bfloat16)
    cid = jax.lax.broadcasted_iota(jnp.int32, (_COLS, _W), 1)
    jid = jax.lax.broadcasted_iota(jnp.int32, (_COLS, _W), 0)
    expand = (cid // _N_CLASSES == jid).astype(jnp.bfloat16)
    xrep = jax.lax.dot_general(
        xf, expand,
        dimension_numbers=(((1,), (0,)), ((), ())),
        preferred_element_type=jnp.float32,
    )
    kconst = (
        jax.lax.broadcasted_iota(jnp.int32, (_BLOCK, _W), 1) % _N_CLASSES
    ).astype(jnp.float32)
    oh = (xrep == kconst).astype(jnp.int32)
    for j in range(_COLS):
        o_ref[j, :, :] = oh[:, j * _N_CLASSES:(j + 1) * _N_CLASSES]


def kernel(x):
    grid = _ROWS // _BLOCK
    out_t = pl.pallas_call(
        _onehot_body,
        grid=(grid,),
        in_specs=[pl.BlockSpec((_BLOCK, _COLS), lambda i: (i, 0))],
        out_specs=pl.BlockSpec((_COLS, _BLOCK, _N_CLASSES), lambda i: (0, i, 0)),
        out_shape=jax.ShapeDtypeStruct((_COLS, _ROWS, _N_CLASSES), jnp.int32),
    )(x)
    return jnp.transpose(out_t, (1, 0, 2))
